# baseline (speedup 1.0000x reference)
import functools

import jax
import jax.numpy as jnp
from jax import lax
from jax.experimental import pallas as pl
from jax.experimental.pallas import tpu as pltpu
from jax.experimental.pallas import tpu_sc as plsc

_NC, _NS = 2, 16
_NW = _NC * _NS
_CS = 16
_L = 16
_NXB = 4
_NPB = 2


@functools.lru_cache(maxsize=None)
def _build_sc_kernel(B, S, D):
    s_per_w = S // _NW
    n_chunks = s_per_w // _CS
    n_groups = n_chunks // 2
    ipg = 2 * B
    assert B == 4 and s_per_w % _CS == 0 and n_chunks % 2 == 0 and n_groups >= 3

    mesh = plsc.VectorSubcoreMesh(
        core_axis_name="c", subcore_axis_name="s",
        num_cores=_NC, num_subcores=_NS,
    )

    @functools.partial(
        pl.kernel,
        mesh=mesh,
        out_type=jax.ShapeDtypeStruct((B, S, D), jnp.float32),
        scratch_types=[
            [pltpu.VMEM((_CS, D), jnp.float32) for _ in range(_NXB)],
            [pltpu.VMEM((_CS, D), jnp.float32) for _ in range(_NPB)],
            pltpu.SemaphoreType.DMA((_NXB,)),
            pltpu.SemaphoreType.DMA((_NXB,)),
            pltpu.SemaphoreType.DMA((_NPB,)),
        ],
    )
    def sc_kernel(x_hbm, pe_hbm, out_hbm, xb, peb, xldsem, xstsem, pesem):
        wid = lax.axis_index("s") * _NC + lax.axis_index("c")
        base = wid * s_per_w

        def x_slice(i_chunk, b):
            return x_hbm.at[b, pl.ds(base + i_chunk * _CS, _CS)]

        def out_slice(i_chunk, b):
            return out_hbm.at[b, pl.ds(base + i_chunk * _CS, _CS)]

        def pe_slice(i_chunk):
            return pe_hbm.at[pl.ds(base + i_chunk * _CS, _CS)]

        def compute(xbuf, pebuf):
            def col_body(j, carry):
                sl = pl.ds(j * _L, _L)
                for r in range(_CS):
                    plsc.addupdate(xbuf.at[r, sl], pebuf[r, sl])
                return carry

            lax.fori_loop(0, D // _L, col_body, 0)

        def emit_group(g, first, last):
            for k in range(ipg):
                c_off = k // B
                b = k % B
                c = 2 * g + c_off
                xi = k % _NXB
                ni = (k + 2) % _NXB
                pltpu.make_async_copy(x_slice(c, b), xb[xi], xldsem.at[xi]).wait()
                if not (first and k < 2):
                    pltpu.make_async_copy(
                        xb[ni], out_slice(2 * g + (k - 2) // B, (k - 2) % B),
                        xstsem.at[ni]).wait()
                if not (last and k >= ipg - 2):
                    nc = 2 * g + ((k + 2) // B)
                    pltpu.async_copy(x_slice(nc, (k + 2) % B), xb[ni], xldsem.at[ni])
                if b == 0:
                    pltpu.make_async_copy(pe_slice(c), peb[c_off], pesem.at[c_off]).wait()
                    if not (last and c_off == 1):
                        pltpu.async_copy(
                            pe_slice(c + 1), peb[(c_off + 1) % _NPB],
                            pesem.at[(c_off + 1) % _NPB])
                compute(xb[xi], peb[c_off])
                pltpu.async_copy(xb[xi], out_slice(c, b), xstsem.at[xi])

        pltpu.async_copy(x_slice(0, 0), xb[0], xldsem.at[0])
        pltpu.async_copy(x_slice(0, 1), xb[1], xldsem.at[1])
        pltpu.async_copy(pe_slice(0), peb[0], pesem.at[0])

        emit_group(0, first=True, last=False)

        def group_body(g, carry):
            emit_group(g, first=False, last=False)
            return carry

        lax.fori_loop(1, n_groups - 1, group_body, 0)

        emit_group(n_groups - 1, first=False, last=True)

        last_chunk = n_chunks - 1
        pltpu.make_async_copy(xb[2], out_slice(last_chunk, B - 2), xstsem.at[2]).wait()
        pltpu.make_async_copy(xb[3], out_slice(last_chunk, B - 1), xstsem.at[3]).wait()

    return sc_kernel


def kernel(x, pe):
    B, S, D = x.shape
    return _build_sc_kernel(B, S, D)(x, pe)


# speedup vs baseline: 1.9009x; 1.9009x over previous
import functools

import jax
import jax.numpy as jnp
from jax import lax
from jax.experimental import pallas as pl
from jax.experimental.pallas import tpu as pltpu
from jax.experimental.pallas import tpu_sc as plsc

_NC, _NS = 2, 16
_NW = _NC * _NS
_CS = 16
_L = 16
_NXB = 4
_NPB = 2


@functools.lru_cache(maxsize=None)
def _build_sc_kernel(B, S, D):
    s_per_w = S // _NW
    n_chunks = s_per_w // _CS
    n_groups = n_chunks // 2
    ipg = 2 * B
    assert B == 4 and s_per_w % _CS == 0 and n_chunks % 2 == 0 and n_groups >= 3

    mesh = plsc.VectorSubcoreMesh(
        core_axis_name="c", subcore_axis_name="s",
        num_cores=_NC, num_subcores=_NS,
    )

    @functools.partial(
        pl.kernel,
        mesh=mesh,
        out_type=jax.ShapeDtypeStruct((B, S, D), jnp.float32),
        scratch_types=[
            [pltpu.VMEM((_CS, D), jnp.float32) for _ in range(_NXB)],
            [pltpu.VMEM((_CS, D), jnp.float32) for _ in range(_NPB)],
            pltpu.SemaphoreType.DMA((_NXB,)),
            pltpu.SemaphoreType.DMA((_NXB,)),
            pltpu.SemaphoreType.DMA((_NPB,)),
        ],
    )
    def sc_kernel(x_hbm, pe_hbm, out_hbm, xb, peb, xldsem, xstsem, pesem):
        wid = lax.axis_index("s") * _NC + lax.axis_index("c")
        base = wid * s_per_w

        def x_slice(i_chunk, b):
            return x_hbm.at[b, pl.ds(base + i_chunk * _CS, _CS)]

        def out_slice(i_chunk, b):
            return out_hbm.at[b, pl.ds(base + i_chunk * _CS, _CS)]

        def pe_slice(i_chunk):
            return pe_hbm.at[pl.ds(base + i_chunk * _CS, _CS)]

        def compute(xbuf, pebuf):
            @plsc.parallel_loop(0, D // _L, unroll=4)
            def col_body(j):
                sl = pl.ds(j * _L, _L)
                for r in range(_CS):
                    plsc.addupdate(xbuf.at[r, sl], pebuf[r, sl])

        def emit_group(g, first, last):
            for k in range(ipg):
                c_off = k // B
                b = k % B
                c = 2 * g + c_off
                xi = k % _NXB
                ni = (k + 2) % _NXB
                pltpu.make_async_copy(x_slice(c, b), xb[xi], xldsem.at[xi]).wait()
                if not (first and k < 2):
                    pltpu.make_async_copy(
                        xb[ni], out_slice(2 * g + (k - 2) // B, (k - 2) % B),
                        xstsem.at[ni]).wait()
                if not (last and k >= ipg - 2):
                    nc = 2 * g + ((k + 2) // B)
                    pltpu.async_copy(x_slice(nc, (k + 2) % B), xb[ni], xldsem.at[ni])
                if b == 0:
                    pltpu.make_async_copy(pe_slice(c), peb[c_off], pesem.at[c_off]).wait()
                    if not (last and c_off == 1):
                        pltpu.async_copy(
                            pe_slice(c + 1), peb[(c_off + 1) % _NPB],
                            pesem.at[(c_off + 1) % _NPB])
                compute(xb[xi], peb[c_off])
                pltpu.async_copy(xb[xi], out_slice(c, b), xstsem.at[xi])

        pltpu.async_copy(x_slice(0, 0), xb[0], xldsem.at[0])
        pltpu.async_copy(x_slice(0, 1), xb[1], xldsem.at[1])
        pltpu.async_copy(pe_slice(0), peb[0], pesem.at[0])

        emit_group(0, first=True, last=False)

        def group_body(g, carry):
            emit_group(g, first=False, last=False)
            return carry

        lax.fori_loop(1, n_groups - 1, group_body, 0)

        emit_group(n_groups - 1, first=False, last=True)

        last_chunk = n_chunks - 1
        pltpu.make_async_copy(xb[2], out_slice(last_chunk, B - 2), xstsem.at[2]).wait()
        pltpu.make_async_copy(xb[3], out_slice(last_chunk, B - 1), xstsem.at[3]).wait()

    return sc_kernel


def kernel(x, pe):
    B, S, D = x.shape
    return _build_sc_kernel(B, S, D)(x, pe)


# speedup vs baseline: 2.7522x; 1.4478x over previous
import jax
import jax.numpy as jnp
from jax.experimental import pallas as pl
from jax.experimental.pallas import tpu as pltpu

_BS = 2048


def _add_pe_kernel(x_ref, pe_ref, o_ref):
    o_ref[...] = x_ref[...] + pe_ref[...][None, :, :]


def kernel(x, pe):
    B, S, D = x.shape
    grid = (S // _BS, B)
    return pl.pallas_call(
        _add_pe_kernel,
        grid=grid,
        in_specs=[
            pl.BlockSpec((1, _BS, D), lambda i, j: (j, i, 0)),
            pl.BlockSpec((_BS, D), lambda i, j: (i, 0)),
        ],
        out_specs=pl.BlockSpec((1, _BS, D), lambda i, j: (j, i, 0)),
        out_shape=jax.ShapeDtypeStruct((B, S, D), x.dtype),
        compiler_params=pltpu.CompilerParams(
            dimension_semantics=("parallel", "parallel"),
        ),
    )(x, pe)
